# baseline (speedup 1.0000x reference)
import jax
import jax.numpy as jnp
from jax import lax
from jax.experimental import pallas as pl
from jax.experimental.pallas import tpu as pltpu
from jax.experimental.pallas import tpu_sc as plsc

_CH = 512


def _sc_body(nk, nv, kb, vb, ok, ov, bufext, tb8, srow, outbuf):
    B, H, S, D = nk.shape
    W = kb.shape[2]
    L = 16
    cid = lax.axis_index("c")
    h = lax.axis_index("s")
    base = pl.multiple_of(cid * (W // 2 - 8), 8)

    def work(src_new, src_buf, dst):
        for c in range(2):
            pltpu.sync_copy(src_buf.at[0, h, pl.ds(base + c * _CH, _CH + 8), :],
                            bufext.at[pl.ds(0, _CH + 8), :])
            for b in range(B):
                pltpu.sync_copy(bufext.at[pl.ds(1, _CH), :],
                                dst.at[b, h, pl.ds(base + c * _CH, _CH), :])
        pltpu.sync_copy(src_buf.at[0, h, pl.ds(W - 8, 8), :], tb8)
        for r in range(7):
            for j in range(D // L):
                outbuf[r, pl.ds(j * L, L)] = tb8[r + 1, pl.ds(j * L, L)]
        for b in range(B):
            pltpu.sync_copy(src_new.at[b, h, :, :], srow)
            for j in range(D // L):
                outbuf[7, pl.ds(j * L, L)] = srow[S - 1, pl.ds(j * L, L)]
            pltpu.sync_copy(outbuf, dst.at[b, h, pl.ds(W - 8, 8), :])

    work(nk, kb, ok)
    work(nv, vb, ov)


def kernel(new_k, new_v, k_buf, v_buf):
    B, H, S, D = new_k.shape
    W = k_buf.shape[2]
    out = jax.ShapeDtypeStruct((B, H, W, D), new_k.dtype)
    mesh = plsc.VectorSubcoreMesh(core_axis_name="c", subcore_axis_name="s")
    f = pl.kernel(
        _sc_body,
        out_type=[out, out],
        mesh=mesh,
        scratch_types=[
            pltpu.VMEM((_CH + 8, D), jnp.float32),
            pltpu.VMEM((8, D), jnp.float32),
            pltpu.VMEM((S, D), jnp.float32),
            pltpu.VMEM((8, D), jnp.float32),
        ],
    )
    updated_k, updated_v = f(new_k, new_v, k_buf, v_buf)
    return (updated_k, updated_v)


# speedup vs baseline: 1.0623x; 1.0623x over previous
import jax
import jax.numpy as jnp
from jax import lax
from jax.experimental import pallas as pl
from jax.experimental.pallas import tpu as pltpu
from jax.experimental.pallas import tpu_sc as plsc

_CH = 512


def _sc_body(nk, kb, ok, bufext, tb8, srow, outbuf):
    B, H, S, D = nk.shape
    W = kb.shape[2]
    L = 16
    cid = lax.axis_index("c")
    h = lax.axis_index("s")
    base = pl.multiple_of(cid * (W // 2 - 8), 8)

    for c in range(2):
        pltpu.sync_copy(kb.at[0, h, pl.ds(base + c * _CH, _CH + 8), :],
                        bufext.at[pl.ds(0, _CH + 8), :])
        for b in range(B):
            pltpu.sync_copy(bufext.at[pl.ds(1, _CH), :],
                            ok.at[b, h, pl.ds(base + c * _CH, _CH), :])
    pltpu.sync_copy(kb.at[0, h, pl.ds(W - 8, 8), :], tb8)
    for r in range(7):
        for j in range(D // L):
            outbuf[r, pl.ds(j * L, L)] = tb8[r + 1, pl.ds(j * L, L)]
    for b in range(B):
        pltpu.sync_copy(nk.at[b, h, :, :], srow)
        for j in range(D // L):
            outbuf[7, pl.ds(j * L, L)] = srow[S - 1, pl.ds(j * L, L)]
        pltpu.sync_copy(outbuf, ok.at[b, h, pl.ds(W - 8, 8), :])


def _tc_body(nv_ref, vb_ref, ov_ref):
    B = ov_ref.shape[0]
    W = ov_ref.shape[2]
    D = ov_ref.shape[3]
    S = nv_ref.shape[2]
    shifted_v = vb_ref[:, :, 1:, :]
    ov_ref[:, :, : W - 1, :] = jnp.broadcast_to(shifted_v, (B, 1, W - 1, D))
    ov_ref[:, :, W - 1 :, :] = nv_ref[:, :, S - 1 :, :]


def kernel(new_k, new_v, k_buf, v_buf):
    B, H, S, D = new_k.shape
    W = k_buf.shape[2]
    out = jax.ShapeDtypeStruct((B, H, W, D), new_k.dtype)

    mesh = plsc.VectorSubcoreMesh(core_axis_name="c", subcore_axis_name="s")
    sc_fn = pl.kernel(
        _sc_body,
        out_type=out,
        mesh=mesh,
        scratch_types=[
            pltpu.VMEM((_CH + 8, D), jnp.float32),
            pltpu.VMEM((8, D), jnp.float32),
            pltpu.VMEM((S, D), jnp.float32),
            pltpu.VMEM((8, D), jnp.float32),
        ],
    )
    updated_k = sc_fn(new_k, k_buf)

    updated_v = pl.pallas_call(
        _tc_body,
        grid=(H,),
        in_specs=[
            pl.BlockSpec((B, 1, S, D), lambda h: (0, h, 0, 0)),
            pl.BlockSpec((1, 1, W, D), lambda h: (0, h, 0, 0)),
        ],
        out_specs=pl.BlockSpec((B, 1, W, D), lambda h: (0, h, 0, 0)),
        out_shape=out,
    )(new_v, v_buf)
    return (updated_k, updated_v)


# speedup vs baseline: 1.1043x; 1.0395x over previous
import jax
import jax.numpy as jnp
from jax import lax
from jax.experimental import pallas as pl
from jax.experimental.pallas import tpu as pltpu
from jax.experimental.pallas import tpu_sc as plsc

_CH = 256
_NC = 4


def _sc_body(nk, kb, ok, buf0, buf1, tb8, srow, outbuf,
             sem_l, sem_w0, sem_w1):
    B, H, S, D = nk.shape
    W = kb.shape[2]
    L = 16
    cid = lax.axis_index("c")
    h = lax.axis_index("s")
    base = pl.multiple_of(cid * (W // 2 - 8), 8)
    bufs = (buf0, buf1)
    wsems = (sem_w0, sem_w1)

    loads = [None] * _NC
    writes = [[] for _ in range(_NC)]
    loads[0] = pltpu.async_copy(
        kb.at[0, h, pl.ds(base, _CH + 8), :],
        bufs[0].at[pl.ds(0, _CH + 8), :], sem_l)
    for c in range(_NC):
        buf = bufs[c % 2]
        loads[c].wait()
        if c + 1 < _NC:
            if c - 1 >= 0:
                for w in writes[c - 1]:
                    w.wait()
            loads[c + 1] = pltpu.async_copy(
                kb.at[0, h, pl.ds(base + (c + 1) * _CH, _CH + 8), :],
                bufs[(c + 1) % 2].at[pl.ds(0, _CH + 8), :], sem_l)
        for b in range(B):
            writes[c].append(pltpu.async_copy(
                buf.at[pl.ds(1, _CH), :],
                ok.at[b, h, pl.ds(base + c * _CH, _CH), :],
                wsems[c % 2]))

    pltpu.sync_copy(kb.at[0, h, pl.ds(W - 8, 8), :], tb8)
    for r in range(7):
        for j in range(D // L):
            outbuf[r, pl.ds(j * L, L)] = tb8[r + 1, pl.ds(j * L, L)]
    for b in range(B):
        pltpu.sync_copy(nk.at[b, h, :, :], srow)
        for j in range(D // L):
            outbuf[7, pl.ds(j * L, L)] = srow[S - 1, pl.ds(j * L, L)]
        pltpu.sync_copy(outbuf, ok.at[b, h, pl.ds(W - 8, 8), :])

    for c in (_NC - 2, _NC - 1):
        for w in writes[c]:
            w.wait()


def _tc_body(nv_ref, vb_ref, ov_ref):
    B = ov_ref.shape[0]
    W = ov_ref.shape[2]
    D = ov_ref.shape[3]
    S = nv_ref.shape[2]
    shifted_v = vb_ref[:, :, 1:, :]
    ov_ref[:, :, : W - 1, :] = jnp.broadcast_to(shifted_v, (B, 1, W - 1, D))
    ov_ref[:, :, W - 1 :, :] = nv_ref[:, :, S - 1 :, :]


def kernel(new_k, new_v, k_buf, v_buf):
    B, H, S, D = new_k.shape
    W = k_buf.shape[2]
    out = jax.ShapeDtypeStruct((B, H, W, D), new_k.dtype)

    mesh = plsc.VectorSubcoreMesh(core_axis_name="c", subcore_axis_name="s")
    sc_fn = pl.kernel(
        _sc_body,
        out_type=out,
        mesh=mesh,
        scratch_types=[
            pltpu.VMEM((_CH + 8, D), jnp.float32),
            pltpu.VMEM((_CH + 8, D), jnp.float32),
            pltpu.VMEM((8, D), jnp.float32),
            pltpu.VMEM((S, D), jnp.float32),
            pltpu.VMEM((8, D), jnp.float32),
            pltpu.SemaphoreType.DMA,
            pltpu.SemaphoreType.DMA,
            pltpu.SemaphoreType.DMA,
        ],
    )
    updated_k = sc_fn(new_k, k_buf)

    updated_v = pl.pallas_call(
        _tc_body,
        grid=(H,),
        in_specs=[
            pl.BlockSpec((B, 1, S, D), lambda h: (0, h, 0, 0)),
            pl.BlockSpec((1, 1, W, D), lambda h: (0, h, 0, 0)),
        ],
        out_specs=pl.BlockSpec((B, 1, W, D), lambda h: (0, h, 0, 0)),
        out_shape=out,
    )(new_v, v_buf)
    return (updated_k, updated_v)
